# baseline (speedup 1.0000x reference)
import functools
import jax
import jax.numpy as jnp
from jax import lax
from jax.experimental import pallas as pl
from jax.experimental.pallas import tpu as pltpu
from jax.experimental.pallas import tpu_sc as plsc

NF = 128
H = 200
W = 200
NW = 32


def _sc_pos_enc(col_hbm, rowflat_hbm, out_hbm, colstage_v, rowrep_v, rowvec_v,
                sem):
    wid = lax.axis_index("s") * 2 + lax.axis_index("c")

    pltpu.sync_copy(col_hbm, colstage_v)

    nrows_max = (H + NW - 1) // NW

    def do_row(t, carry):
        i = jnp.minimum(t * NW + wid, H - 1)

        pltpu.sync_copy(rowflat_hbm.at[pl.ds(i * NF, NF)], rowvec_v)
        for q in range(NF // 16):
            chunk = rowvec_v[pl.ds(q * 16, 16)]
            for r in range(8):
                rowrep_v[r, pl.ds(q * 16, 16)] = chunk
        pix0 = i * W

        def group(g, inner):
            p = pix0 + g * 8
            pltpu.sync_copy(colstage_v.at[pl.ds(g * 8, 8)],
                            out_hbm.at[pl.ds(p, 8), pl.ds(0, NF)])
            pltpu.sync_copy(rowrep_v,
                            out_hbm.at[pl.ds(p, 8), pl.ds(NF, NF)])
            return inner

        lax.fori_loop(0, W // 8, group, 0)

        return carry

    lax.fori_loop(0, nrows_max, do_row, 0)


def kernel(bev_mask, row_embed, col_embed):
    b = bev_mask.shape[0]
    h, w = bev_mask.shape[-2], bev_mask.shape[-1]
    nf = row_embed.shape[1]

    mesh = plsc.VectorSubcoreMesh(core_axis_name="c", subcore_axis_name="s")
    run = functools.partial(
        pl.kernel,
        mesh=mesh,
        out_type=jax.ShapeDtypeStruct((h * w, 2 * nf), jnp.float32),
        scratch_types=[
            pltpu.VMEM((w, nf), jnp.float32),
            pltpu.VMEM((8, nf), jnp.float32),
            pltpu.VMEM((nf,), jnp.float32),
            pltpu.SemaphoreType.DMA,
        ],
    )(_sc_pos_enc)
    pos2d = run(col_embed[:w], row_embed[:h].reshape(-1))
    out = jnp.transpose(pos2d.reshape(h, w, 2 * nf), (2, 0, 1))[None]
    return jnp.broadcast_to(out, (b, 2 * nf, h, w))


# speedup vs baseline: 1.4072x; 1.4072x over previous
import functools
import jax
import jax.numpy as jnp
from jax import lax
from jax.experimental import pallas as pl
from jax.experimental.pallas import tpu as pltpu
from jax.experimental.pallas import tpu_sc as plsc

NF = 128
H = 200
W = 200
NW = 32


def _sc_pos_enc(col_hbm, rowflat_hbm, out_hbm, colstage_v, rowrep_v, rowvec_v,
                sem):
    wid = lax.axis_index("s") * 2 + lax.axis_index("c")

    pltpu.sync_copy(col_hbm, colstage_v)

    nrows_max = (H + NW - 1) // NW

    def do_row(t, carry):
        i = jnp.minimum(t * NW + wid, H - 1)

        pltpu.sync_copy(rowflat_hbm.at[pl.ds(i * NF, NF)], rowvec_v)
        for q in range(NF // 16):
            chunk = rowvec_v[pl.ds(q * 16, 16)]
            for r in range(8):
                rowrep_v[r, pl.ds(q * 16, 16)] = chunk
        pix0 = i * W

        def group(g, inner):
            p = pix0 + g * 8
            pltpu.async_copy(colstage_v.at[pl.ds(g * 8, 8)],
                             out_hbm.at[pl.ds(p, 8), pl.ds(0, NF)], sem)
            pltpu.async_copy(rowrep_v,
                             out_hbm.at[pl.ds(p, 8), pl.ds(NF, NF)], sem)
            return inner

        lax.fori_loop(0, W // 8, group, 0)

        def drain(g, inner):
            p = pix0 + g * 8
            pltpu.make_async_copy(
                colstage_v.at[pl.ds(g * 8, 8)],
                out_hbm.at[pl.ds(p, 8), pl.ds(0, NF)], sem).wait()
            pltpu.make_async_copy(
                rowrep_v,
                out_hbm.at[pl.ds(p, 8), pl.ds(NF, NF)], sem).wait()
            return inner

        lax.fori_loop(0, W // 8, drain, 0)

        return carry

    lax.fori_loop(0, nrows_max, do_row, 0)


def kernel(bev_mask, row_embed, col_embed):
    b = bev_mask.shape[0]
    h, w = bev_mask.shape[-2], bev_mask.shape[-1]
    nf = row_embed.shape[1]

    mesh = plsc.VectorSubcoreMesh(core_axis_name="c", subcore_axis_name="s")
    run = functools.partial(
        pl.kernel,
        mesh=mesh,
        out_type=jax.ShapeDtypeStruct((h * w, 2 * nf), jnp.float32),
        scratch_types=[
            pltpu.VMEM((w, nf), jnp.float32),
            pltpu.VMEM((8, nf), jnp.float32),
            pltpu.VMEM((nf,), jnp.float32),
            pltpu.SemaphoreType.DMA,
        ],
    )(_sc_pos_enc)
    pos2d = run(col_embed[:w], row_embed[:h].reshape(-1))
    out = jnp.transpose(pos2d.reshape(h, w, 2 * nf), (2, 0, 1))[None]
    return jnp.broadcast_to(out, (b, 2 * nf, h, w))


# speedup vs baseline: 1.4652x; 1.0412x over previous
import functools
import jax
import jax.numpy as jnp
from jax import lax
from jax.experimental import pallas as pl
from jax.experimental.pallas import tpu as pltpu
from jax.experimental.pallas import tpu_sc as plsc

NF = 128
H = 200
W = 200
NW = 32
NROWS = (H + NW - 1) // NW


def _sc_pos_enc(col_hbm, rowflat_hbm, out_hbm, colstage_v, bufa_v, bufb_v,
                rowvec_v, col_sem, sem_a, sem_b):
    wid = lax.axis_index("s") * 2 + lax.axis_index("c")

    pltpu.sync_copy(col_hbm, colstage_v)

    bufs = (bufa_v, bufb_v)
    sems = (sem_a, sem_b)

    def row_of(t):
        return jnp.minimum(t * NW + wid, H - 1)

    def fill(buf, i):
        pltpu.sync_copy(rowflat_hbm.at[pl.ds(i * NF, NF)], rowvec_v)
        chunks = [rowvec_v[pl.ds(q * 16, 16)] for q in range(NF // 16)]

        def body(r, inner):
            for q in range(NF // 16):
                buf[r, pl.ds(q * 16, 16)] = chunks[q]
            return inner

        lax.fori_loop(0, W, body, 0)

    for t in range(NROWS):
        buf, sem = bufs[t % 2], sems[t % 2]
        if t >= 2:
            iprev = row_of(t - 2)
            pltpu.make_async_copy(
                buf, out_hbm.at[pl.ds(iprev * W, W), pl.ds(NF, NF)],
                sem).wait()
        i = row_of(t)
        fill(buf, i)
        pltpu.async_copy(colstage_v,
                         out_hbm.at[pl.ds(i * W, W), pl.ds(0, NF)], col_sem)
        pltpu.async_copy(buf,
                         out_hbm.at[pl.ds(i * W, W), pl.ds(NF, NF)], sem)

    for t in (NROWS - 2, NROWS - 1):
        i = row_of(t)
        pltpu.make_async_copy(
            bufs[t % 2], out_hbm.at[pl.ds(i * W, W), pl.ds(NF, NF)],
            sems[t % 2]).wait()
    for t in range(NROWS):
        i = row_of(t)
        pltpu.make_async_copy(
            colstage_v, out_hbm.at[pl.ds(i * W, W), pl.ds(0, NF)],
            col_sem).wait()


def kernel(bev_mask, row_embed, col_embed):
    b = bev_mask.shape[0]
    h, w = bev_mask.shape[-2], bev_mask.shape[-1]
    nf = row_embed.shape[1]

    mesh = plsc.VectorSubcoreMesh(core_axis_name="c", subcore_axis_name="s")
    run = functools.partial(
        pl.kernel,
        mesh=mesh,
        out_type=jax.ShapeDtypeStruct((h * w, 2 * nf), jnp.float32),
        scratch_types=[
            pltpu.VMEM((w, nf), jnp.float32),
            pltpu.VMEM((w, nf), jnp.float32),
            pltpu.VMEM((w, nf), jnp.float32),
            pltpu.VMEM((nf,), jnp.float32),
            pltpu.SemaphoreType.DMA,
            pltpu.SemaphoreType.DMA,
            pltpu.SemaphoreType.DMA,
        ],
    )(_sc_pos_enc)
    pos2d = run(col_embed[:w], row_embed[:h].reshape(-1))
    out = jnp.transpose(pos2d.reshape(h, w, 2 * nf), (2, 0, 1))[None]
    return jnp.broadcast_to(out, (b, 2 * nf, h, w))
